# baseline (speedup 1.0000x reference)
import functools

import jax
import jax.numpy as jnp
from jax import lax
from jax.experimental import pallas as pl
from jax.experimental.pallas import tpu as pltpu
from jax.experimental.pallas import tpu_sc as plsc

N_NODES = 10000
N_EDGES = 320000
N_GRAPHS = 64
DIM = 64
N_LAYERS = 5
BN_EPS = 1e-5

NC = 2
NS = 16
CHUNK = 128
K = 160
EW = K * CHUNK
EPAD = NS * EW
ROWS_PER_TILE = 640
NACC = NS * ROWS_PER_TILE
ZROWS = 128


NBUF = 2


def _make_body(sup, ks, zrows, feat_split):
  def _sc_agg_body(x_hbm, src_hbm, dst_hbm, out_hbm, src_v, dst_v, rows_v,
                   zbuf, acc, gsem, ssem):
    c = lax.axis_index("c")
    s = lax.axis_index("s")
    dh = zbuf.shape[1]
    KS = ks
    ZROWS = zrows
    w = s if feat_split else c * NS + s

    def zrow(i, _):
        for cc in range(dh // 16):
            zbuf[i, pl.ds(cc * 16, 16)] = jnp.zeros((16,), jnp.float32)
        return 0
    lax.fori_loop(0, ZROWS, zrow, 0)
    for b in range(ROWS_PER_TILE // ZROWS):
        pltpu.sync_copy(zbuf, acc.at[pl.ds(s * ROWS_PER_TILE + b * ZROWS,
                                           ZROWS)])

    pltpu.sync_copy(src_hbm.at[pl.ds(w * KS, KS)], src_v)
    pltpu.sync_copy(dst_hbm.at[pl.ds(w * KS, KS)], dst_v)
    plsc.subcore_barrier()

    xh = x_hbm.at[c] if feat_split else x_hbm

    def group(g, _):
        base = g * NBUF
        for b in range(NBUF):
            j = base + b

            @pl.when(g > 0)
            def _():
                pltpu.make_async_copy(
                    rows_v.at[b], acc.at[dst_v.at[j - NBUF]],
                    ssem.at[b]).wait()
            pltpu.async_copy(xh.at[src_v.at[j]], rows_v.at[b], gsem.at[b])
        for b in range(NBUF):
            j = base + b
            pltpu.make_async_copy(xh.at[src_v.at[j]], rows_v.at[b],
                                  gsem.at[b]).wait()
            pltpu.async_copy(rows_v.at[b], acc.at[dst_v.at[j]],
                             ssem.at[b], add=True)
        return 0
    lax.fori_loop(0, KS // NBUF, group, 0)
    for b in range(NBUF):
        pltpu.make_async_copy(rows_v.at[b], acc.at[dst_v.at[KS - NBUF + b]],
                              ssem.at[b]).wait()
    plsc.subcore_barrier()

    pltpu.sync_copy(acc.at[pl.ds(s * ROWS_PER_TILE, ROWS_PER_TILE)],
                    out_hbm.at[c].at[pl.ds(s * ROWS_PER_TILE, ROWS_PER_TILE)])
  return _sc_agg_body


@functools.lru_cache(maxsize=None)
def _make_sc_agg(d, feat_split):
    dh = d // NC if feat_split else d
    sup = 2 if dh == 64 and feat_split else 4
    ks = (K if feat_split else K // NC) // sup
    zrows = 32
    mesh = plsc.VectorSubcoreMesh(core_axis_name="c", subcore_axis_name="s")
    return pl.kernel(
        _make_body(sup, ks, zrows, feat_split),
        out_type=jax.ShapeDtypeStruct((NC, NACC, dh), jnp.float32),
        mesh=mesh,
        compiler_params=pltpu.CompilerParams(use_tc_tiling_on_sc=False),
        scratch_types=[
            pltpu.VMEM((ks, sup * CHUNK), jnp.int32),
            pltpu.VMEM((ks, sup * CHUNK), jnp.int32),
            pltpu.VMEM((NBUF, sup * CHUNK, dh), jnp.float32),
            pltpu.VMEM((zrows, dh), jnp.float32),
            pltpu.VMEM_SHARED((NACC, dh), jnp.float32),
            pltpu.SemaphoreType.DMA((NBUF,)),
            pltpu.SemaphoreType.DMA((NBUF,)),
        ],
    )


def _tc_layer_body(x_ref, agg_ref, w1_ref, b1_ref, w2_ref, b2_ref,
                   g_ref, be_ref, gi_ref, h_ref, pool_ref, *, feat_split):
    if feat_split:
        agg = jnp.concatenate(
            [agg_ref[0, :N_NODES, :], agg_ref[1, :N_NODES, :]], axis=1)
    else:
        agg = agg_ref[0, :N_NODES, :] + agg_ref[1, :N_NODES, :]
    u = x_ref[...] + agg
    t = jnp.dot(u, w1_ref[...], preferred_element_type=jnp.float32)
    t = jnp.maximum(t + b1_ref[...], 0.0)
    v = jnp.dot(t, w2_ref[...], preferred_element_type=jnp.float32)
    v = jnp.maximum(v + b2_ref[...], 0.0)
    mean = jnp.mean(v, axis=0, keepdims=True)
    ctr = v - mean
    var = jnp.mean(ctr * ctr, axis=0, keepdims=True)
    hn = (v - mean) / jnp.sqrt(var + BN_EPS) * g_ref[...] + be_ref[...]
    h_ref[...] = hn
    onehot = (lax.broadcasted_iota(jnp.int32, (N_GRAPHS, N_NODES), 0)
              == gi_ref[...]).astype(jnp.float32)
    pool_ref[...] = jnp.dot(onehot, hn, preferred_element_type=jnp.float32,
                            precision=lax.Precision.HIGHEST)


def _tc_layer(x, agg, p, gi2d, feat_split):
    return pl.pallas_call(
        functools.partial(_tc_layer_body, feat_split=feat_split),
        out_shape=[
            jax.ShapeDtypeStruct((N_NODES, DIM), jnp.float32),
            jax.ShapeDtypeStruct((N_GRAPHS, DIM), jnp.float32),
        ],
    )(x, agg, p["W1"], p["b1"].reshape(1, -1), p["W2"],
      p["b2"].reshape(1, -1), p["gamma"].reshape(1, -1),
      p["beta"].reshape(1, -1), gi2d)


def kernel(node_features, edge_index, graph_index, params):
    pad = EPAD - N_EDGES
    src_flat = jnp.concatenate([edge_index[0], jnp.zeros((pad,), jnp.int32)])
    dst_flat = jnp.concatenate(
        [edge_index[1], jnp.full((pad,), N_NODES, jnp.int32)])
    gi2d = graph_index.reshape(1, N_NODES)

    x = node_features
    hs, pools = [], []
    for i in range(N_LAYERS):
        p = params[f"layer_{i}"]
        d = x.shape[1]
        feat_split = d == 128
        sup = 2 if feat_split else 4
        src2d = src_flat.reshape(-1, sup * CHUNK)
        dst2d = dst_flat.reshape(-1, sup * CHUNK)
        if feat_split:
            dh = d // NC
            xin = jnp.stack([x[:, :dh], x[:, dh:]])
        else:
            xin = x
        agg = _make_sc_agg(d, feat_split)(xin, src2d, dst2d)
        h, pool = _tc_layer(x, agg, p, gi2d, feat_split)
        x = h
        hs.append(h)
        pools.append(pool)
    return jnp.concatenate(pools, axis=1), jnp.concatenate(hs, axis=1)


# speedup vs baseline: 1.2506x; 1.2506x over previous
import functools

import jax
import jax.numpy as jnp
from jax import lax
from jax.experimental import pallas as pl
from jax.experimental.pallas import tpu as pltpu
from jax.experimental.pallas import tpu_sc as plsc

N_NODES = 10000
N_EDGES = 320000
N_GRAPHS = 64
DIM = 64
N_LAYERS = 5
BN_EPS = 1e-5

NC = 2
NS = 16
CHUNK = 128
K = 160
EW = K * CHUNK
EPAD = NS * EW
ROWS_PER_TILE = 640
NACC = NS * ROWS_PER_TILE
ZROWS = 128


def _make_body(sup, ks, zrows, feat_split, nbuf):
  def _sc_agg_body(x_hbm, src_hbm, dst_hbm, out_hbm, src_v, dst_v, rows_v,
                   zbuf, acc, gsem, ssem):
    c = lax.axis_index("c")
    s = lax.axis_index("s")
    dh = zbuf.shape[1]
    KS = ks
    ZROWS = zrows
    w = s if feat_split else c * NS + s
    NBUF = nbuf

    def zrow(i, _):
        for cc in range(dh // 16):
            zbuf[i, pl.ds(cc * 16, 16)] = jnp.zeros((16,), jnp.float32)
        return 0
    lax.fori_loop(0, ZROWS, zrow, 0)
    for b in range(ROWS_PER_TILE // ZROWS):
        pltpu.sync_copy(zbuf, acc.at[pl.ds(s * ROWS_PER_TILE + b * ZROWS,
                                           ZROWS)])

    pltpu.sync_copy(src_hbm.at[pl.ds(w * KS, KS)], src_v)
    pltpu.sync_copy(dst_hbm.at[pl.ds(w * KS, KS)], dst_v)
    plsc.subcore_barrier()

    xh = x_hbm.at[c] if feat_split else x_hbm

    def group(g, _):
        base = g * NBUF
        for b in range(NBUF):
            j = base + b

            @pl.when(g > 0)
            def _():
                pltpu.make_async_copy(
                    rows_v.at[b], acc.at[dst_v.at[j - NBUF]],
                    ssem.at[b]).wait()
            pltpu.async_copy(xh.at[src_v.at[j]], rows_v.at[b], gsem.at[b])
        for b in range(NBUF):
            j = base + b
            pltpu.make_async_copy(xh.at[src_v.at[j]], rows_v.at[b],
                                  gsem.at[b]).wait()
            pltpu.async_copy(rows_v.at[b], acc.at[dst_v.at[j]],
                             ssem.at[b], add=True)
        return 0
    lax.fori_loop(0, KS // NBUF, group, 0)
    for b in range(NBUF):
        pltpu.make_async_copy(rows_v.at[b], acc.at[dst_v.at[KS - NBUF + b]],
                              ssem.at[b]).wait()
    plsc.subcore_barrier()

    pltpu.sync_copy(acc.at[pl.ds(s * ROWS_PER_TILE, ROWS_PER_TILE)],
                    out_hbm.at[c].at[pl.ds(s * ROWS_PER_TILE, ROWS_PER_TILE)])
  return _sc_agg_body


@functools.lru_cache(maxsize=None)
def _make_sc_agg(d, feat_split):
    dh = d // NC if feat_split else d
    sup = 2 if dh == 64 else 4
    nbuf = 2 if dh == 64 else 4
    ks = (K if feat_split else K // NC) // sup
    zrows = 32
    mesh = plsc.VectorSubcoreMesh(core_axis_name="c", subcore_axis_name="s")
    return pl.kernel(
        _make_body(sup, ks, zrows, feat_split, nbuf),
        out_type=jax.ShapeDtypeStruct((NC, NACC, dh), jnp.float32),
        mesh=mesh,
        compiler_params=pltpu.CompilerParams(use_tc_tiling_on_sc=False),
        scratch_types=[
            pltpu.VMEM((ks, sup * CHUNK), jnp.int32),
            pltpu.VMEM((ks, sup * CHUNK), jnp.int32),
            pltpu.VMEM((nbuf, sup * CHUNK, dh), jnp.float32),
            pltpu.VMEM((zrows, dh), jnp.float32),
            pltpu.VMEM_SHARED((NACC, dh), jnp.float32),
            pltpu.SemaphoreType.DMA((nbuf,)),
            pltpu.SemaphoreType.DMA((nbuf,)),
        ],
    )


def _tc_layer_body(x_ref, agg_ref, w1_ref, b1_ref, w2_ref, b2_ref,
                   g_ref, be_ref, gi_ref, h_ref, pool_ref, *, feat_split):
    if feat_split:
        agg = jnp.concatenate(
            [agg_ref[0, :N_NODES, :], agg_ref[1, :N_NODES, :]], axis=1)
    else:
        agg = agg_ref[0, :N_NODES, :] + agg_ref[1, :N_NODES, :]
    u = x_ref[...] + agg
    t = jnp.dot(u, w1_ref[...], preferred_element_type=jnp.float32)
    t = jnp.maximum(t + b1_ref[...], 0.0)
    v = jnp.dot(t, w2_ref[...], preferred_element_type=jnp.float32)
    v = jnp.maximum(v + b2_ref[...], 0.0)
    mean = jnp.mean(v, axis=0, keepdims=True)
    ctr = v - mean
    var = jnp.mean(ctr * ctr, axis=0, keepdims=True)
    hn = (v - mean) / jnp.sqrt(var + BN_EPS) * g_ref[...] + be_ref[...]
    h_ref[...] = hn
    onehot = (lax.broadcasted_iota(jnp.int32, (N_GRAPHS, N_NODES), 0)
              == gi_ref[...]).astype(jnp.float32)
    pool_ref[...] = jnp.dot(onehot, hn, preferred_element_type=jnp.float32,
                            precision=lax.Precision.HIGHEST)


def _tc_layer(x, agg, p, gi2d, feat_split):
    return pl.pallas_call(
        functools.partial(_tc_layer_body, feat_split=feat_split),
        out_shape=[
            jax.ShapeDtypeStruct((N_NODES, DIM), jnp.float32),
            jax.ShapeDtypeStruct((N_GRAPHS, DIM), jnp.float32),
        ],
    )(x, agg, p["W1"], p["b1"].reshape(1, -1), p["W2"],
      p["b2"].reshape(1, -1), p["gamma"].reshape(1, -1),
      p["beta"].reshape(1, -1), gi2d)


def kernel(node_features, edge_index, graph_index, params):
    pad = EPAD - N_EDGES
    src_flat = jnp.concatenate([edge_index[0], jnp.zeros((pad,), jnp.int32)])
    dst_flat = jnp.concatenate(
        [edge_index[1], jnp.full((pad,), N_NODES, jnp.int32)])
    gi2d = graph_index.reshape(1, N_NODES)

    x = node_features
    hs, pools = [], []
    for i in range(N_LAYERS):
        p = params[f"layer_{i}"]
        d = x.shape[1]
        feat_split = True
        sup = 2 if d // NC == 64 else 4
        src2d = src_flat.reshape(-1, sup * CHUNK)
        dst2d = dst_flat.reshape(-1, sup * CHUNK)
        if feat_split:
            dh = d // NC
            xin = jnp.stack([x[:, :dh], x[:, dh:]])
        else:
            xin = x
        agg = _make_sc_agg(d, feat_split)(xin, src2d, dst2d)
        h, pool = _tc_layer(x, agg, p, gi2d, feat_split)
        x = h
        hs.append(h)
        pools.append(pool)
    return jnp.concatenate(pools, axis=1), jnp.concatenate(hs, axis=1)


# speedup vs baseline: 1.5379x; 1.2298x over previous
import functools

import jax
import jax.numpy as jnp
from jax import lax
from jax.experimental import pallas as pl
from jax.experimental.pallas import tpu as pltpu
from jax.experimental.pallas import tpu_sc as plsc

N_NODES = 10000
N_EDGES = 320000
N_GRAPHS = 64
DIM = 64
N_LAYERS = 5
BN_EPS = 1e-5

NC = 2
NS = 16
CHUNK = 128
K = 160
EW = K * CHUNK
EPAD = NS * EW
ROWS_PER_TILE = 640
NACC = NS * ROWS_PER_TILE
ZROWS = 128


def _make_body(sup, ks, zrows, feat_split, nbuf, spmem_src):
  def _sc_agg_body(x_hbm, src_hbm, dst_hbm, out_hbm, src_v, dst_v, rows_v,
                   zbuf, acc, xs_sp, gsem, ssem):
    c = lax.axis_index("c")
    s = lax.axis_index("s")
    dh = zbuf.shape[1]
    KS = ks
    ZROWS = zrows
    w = s if feat_split else c * NS + s
    NBUF = nbuf

    def zrow(i, _):
        for cc in range(dh // 16):
            zbuf[i, pl.ds(cc * 16, 16)] = jnp.zeros((16,), jnp.float32)
        return 0
    lax.fori_loop(0, ZROWS, zrow, 0)
    for b in range(ROWS_PER_TILE // ZROWS):
        pltpu.sync_copy(zbuf, acc.at[pl.ds(s * ROWS_PER_TILE + b * ZROWS,
                                           ZROWS)])

    pltpu.sync_copy(src_hbm.at[pl.ds(w * KS, KS)], src_v)
    pltpu.sync_copy(dst_hbm.at[pl.ds(w * KS, KS)], dst_v)
    if spmem_src:
        pltpu.sync_copy(x_hbm.at[c].at[pl.ds(s * ROWS_PER_TILE,
                                             ROWS_PER_TILE)],
                        xs_sp.at[pl.ds(s * ROWS_PER_TILE, ROWS_PER_TILE)])
    plsc.subcore_barrier()

    xh = xs_sp if spmem_src else (x_hbm.at[c] if feat_split else x_hbm)

    def group(g, _):
        base = g * NBUF
        for b in range(NBUF):
            j = base + b

            @pl.when(g > 0)
            def _():
                pltpu.make_async_copy(
                    rows_v.at[b], acc.at[dst_v.at[j - NBUF]],
                    ssem.at[b]).wait()
            pltpu.async_copy(xh.at[src_v.at[j]], rows_v.at[b], gsem.at[b])
        for b in range(NBUF):
            j = base + b
            pltpu.make_async_copy(xh.at[src_v.at[j]], rows_v.at[b],
                                  gsem.at[b]).wait()
            pltpu.async_copy(rows_v.at[b], acc.at[dst_v.at[j]],
                             ssem.at[b], add=True)
        return 0
    lax.fori_loop(0, KS // NBUF, group, 0)
    for b in range(NBUF):
        pltpu.make_async_copy(rows_v.at[b], acc.at[dst_v.at[KS - NBUF + b]],
                              ssem.at[b]).wait()
    plsc.subcore_barrier()

    pltpu.sync_copy(acc.at[pl.ds(s * ROWS_PER_TILE, ROWS_PER_TILE)],
                    out_hbm.at[c].at[pl.ds(s * ROWS_PER_TILE, ROWS_PER_TILE)])
  return _sc_agg_body


@functools.lru_cache(maxsize=None)
def _make_sc_agg(d, feat_split):
    dh = d // NC if feat_split else d
    sup = 2 if dh == 64 else 4
    spmem_src = dh == 32
    nbuf = 2
    ks = (K if feat_split else K // NC) // sup
    zrows = 32
    mesh = plsc.VectorSubcoreMesh(core_axis_name="c", subcore_axis_name="s")
    return pl.kernel(
        _make_body(sup, ks, zrows, feat_split, nbuf, spmem_src),
        out_type=jax.ShapeDtypeStruct((NC, NACC, dh), jnp.float32),
        mesh=mesh,
        compiler_params=pltpu.CompilerParams(use_tc_tiling_on_sc=False),
        scratch_types=[
            pltpu.VMEM((ks, sup * CHUNK), jnp.int32),
            pltpu.VMEM((ks, sup * CHUNK), jnp.int32),
            pltpu.VMEM((nbuf, sup * CHUNK, dh), jnp.float32),
            pltpu.VMEM((zrows, dh), jnp.float32),
            pltpu.VMEM_SHARED((NACC, dh), jnp.float32),
            pltpu.VMEM_SHARED((NACC if spmem_src else 8, dh), jnp.float32),
            pltpu.SemaphoreType.DMA((nbuf,)),
            pltpu.SemaphoreType.DMA((nbuf,)),
        ],
    )


def _tc_layer_body(x_ref, agg_ref, w1_ref, b1_ref, w2_ref, b2_ref,
                   g_ref, be_ref, gi_ref, h_ref, pool_ref, *, feat_split):
    if feat_split:
        agg = jnp.concatenate(
            [agg_ref[0, :N_NODES, :], agg_ref[1, :N_NODES, :]], axis=1)
    else:
        agg = agg_ref[0, :N_NODES, :] + agg_ref[1, :N_NODES, :]
    u = x_ref[...] + agg
    t = jnp.dot(u, w1_ref[...], preferred_element_type=jnp.float32)
    t = jnp.maximum(t + b1_ref[...], 0.0)
    v = jnp.dot(t, w2_ref[...], preferred_element_type=jnp.float32)
    v = jnp.maximum(v + b2_ref[...], 0.0)
    mean = jnp.mean(v, axis=0, keepdims=True)
    ctr = v - mean
    var = jnp.mean(ctr * ctr, axis=0, keepdims=True)
    hn = (v - mean) / jnp.sqrt(var + BN_EPS) * g_ref[...] + be_ref[...]
    h_ref[...] = hn
    onehot = (lax.broadcasted_iota(jnp.int32, (N_GRAPHS, N_NODES), 0)
              == gi_ref[...]).astype(jnp.float32)
    pool_ref[...] = jnp.dot(onehot, hn, preferred_element_type=jnp.float32,
                            precision=lax.Precision.HIGHEST)


def _tc_layer(x, agg, p, gi2d, feat_split):
    return pl.pallas_call(
        functools.partial(_tc_layer_body, feat_split=feat_split),
        out_shape=[
            jax.ShapeDtypeStruct((N_NODES, DIM), jnp.float32),
            jax.ShapeDtypeStruct((N_GRAPHS, DIM), jnp.float32),
        ],
    )(x, agg, p["W1"], p["b1"].reshape(1, -1), p["W2"],
      p["b2"].reshape(1, -1), p["gamma"].reshape(1, -1),
      p["beta"].reshape(1, -1), gi2d)


def kernel(node_features, edge_index, graph_index, params):
    pad = EPAD - N_EDGES
    src_flat = jnp.concatenate([edge_index[0], jnp.zeros((pad,), jnp.int32)])
    dst_flat = jnp.concatenate(
        [edge_index[1], jnp.full((pad,), N_NODES, jnp.int32)])
    gi2d = graph_index.reshape(1, N_NODES)

    x = node_features
    hs, pools = [], []
    for i in range(N_LAYERS):
        p = params[f"layer_{i}"]
        d = x.shape[1]
        feat_split = True
        sup = 2 if d // NC == 64 else 4
        src2d = src_flat.reshape(-1, sup * CHUNK)
        dst2d = dst_flat.reshape(-1, sup * CHUNK)
        if feat_split:
            dh = d // NC
            xin = jnp.stack([x[:, :dh], x[:, dh:]])
            if dh == 32:
                xin = jnp.concatenate(
                    [xin, jnp.zeros((NC, NACC - N_NODES, dh), jnp.float32)],
                    axis=1)
        else:
            xin = x
        agg = _make_sc_agg(d, feat_split)(xin, src2d, dst2d)
        h, pool = _tc_layer(x, agg, p, gi2d, feat_split)
        x = h
        hs.append(h)
        pools.append(pool)
    return jnp.concatenate(pools, axis=1), jnp.concatenate(hs, axis=1)


# speedup vs baseline: 1.7482x; 1.1367x over previous
import functools

import jax
import jax.numpy as jnp
from jax import lax
from jax.experimental import pallas as pl
from jax.experimental.pallas import tpu as pltpu
from jax.experimental.pallas import tpu_sc as plsc

N_NODES = 10000
N_EDGES = 320000
N_GRAPHS = 64
DIM = 64
N_LAYERS = 5
BN_EPS = 1e-5

NC = 2
NS = 16
CHUNK = 128
K = 160
EW = K * CHUNK
EPAD = NS * EW
ROWS_PER_TILE = 640
NACC = NS * ROWS_PER_TILE
ZROWS = 128


def _make_body(sup, ks, zrows, feat_split, nbuf, spmem_src):
  def _sc_agg_body(x_hbm, src_hbm, dst_hbm, out_hbm, src_v, dst_v, rows_v,
                   zbuf, acc, xs_sp, gsem, ssem):
    c = lax.axis_index("c")
    s = lax.axis_index("s")
    dh = zbuf.shape[1]
    KS = ks
    ZROWS = zrows
    w = s if feat_split else c * NS + s
    NBUF = nbuf

    def zrow(i, _):
        for cc in range(dh // 16):
            zbuf[i, pl.ds(cc * 16, 16)] = jnp.zeros((16,), jnp.float32)
        return 0
    lax.fori_loop(0, ZROWS, zrow, 0)
    for b in range(ROWS_PER_TILE // ZROWS):
        pltpu.sync_copy(zbuf, acc.at[pl.ds(s * ROWS_PER_TILE + b * ZROWS,
                                           ZROWS)])

    pltpu.sync_copy(src_hbm.at[pl.ds(w * KS, KS)], src_v)
    pltpu.sync_copy(dst_hbm.at[pl.ds(w * KS, KS)], dst_v)
    if spmem_src:
        pltpu.sync_copy(x_hbm.at[c].at[pl.ds(s * ROWS_PER_TILE,
                                             ROWS_PER_TILE)],
                        xs_sp.at[pl.ds(s * ROWS_PER_TILE, ROWS_PER_TILE)])
    plsc.subcore_barrier()

    xh = xs_sp if spmem_src else (x_hbm.at[c] if feat_split else x_hbm)

    def group(g, _):
        base = g * NBUF
        for b in range(NBUF):
            j = base + b

            @pl.when(g > 0)
            def _():
                pltpu.make_async_copy(
                    rows_v.at[b], acc.at[dst_v.at[j - NBUF]],
                    ssem.at[b]).wait()
            pltpu.async_copy(xh.at[src_v.at[j]], rows_v.at[b], gsem.at[b])
        for b in range(NBUF):
            j = base + b
            pltpu.make_async_copy(xh.at[src_v.at[j]], rows_v.at[b],
                                  gsem.at[b]).wait()
            pltpu.async_copy(rows_v.at[b], acc.at[dst_v.at[j]],
                             ssem.at[b], add=True)
        return 0
    lax.fori_loop(0, KS // NBUF, group, 0)
    for b in range(NBUF):
        pltpu.make_async_copy(rows_v.at[b], acc.at[dst_v.at[KS - NBUF + b]],
                              ssem.at[b]).wait()
    plsc.subcore_barrier()

    pltpu.sync_copy(acc.at[pl.ds(s * ROWS_PER_TILE, ROWS_PER_TILE)],
                    out_hbm.at[c].at[pl.ds(s * ROWS_PER_TILE, ROWS_PER_TILE)])
  return _sc_agg_body


def _make_body_quad(sup, ks, zrows, nbuf):
  def _sc_agg_body(x_hbm, src_hbm, dst_hbm, out_hbm, src_v, dst_v, rows_v,
                   zbuf, acc, xs_sp, gsem, ssem):
    c = lax.axis_index("c")
    s = lax.axis_index("s")
    dh = zbuf.shape[1]
    KS = ks
    ZROWS = zrows
    NBUF = nbuf

    def zrow(i, _):
        for cc in range(dh // 16):
            zbuf[i, pl.ds(cc * 16, 16)] = jnp.zeros((16,), jnp.float32)
        return 0
    lax.fori_loop(0, ZROWS, zrow, 0)

    pltpu.sync_copy(src_hbm.at[pl.ds(s * KS, KS)], src_v)
    pltpu.sync_copy(dst_hbm.at[pl.ds(s * KS, KS)], dst_v)

    for q in range(2):
        qi = 2 * c + q
        for b in range(ROWS_PER_TILE // ZROWS):
            pltpu.sync_copy(zbuf,
                            acc.at[pl.ds(s * ROWS_PER_TILE + b * ZROWS,
                                         ZROWS)])
        pltpu.sync_copy(x_hbm.at[qi].at[pl.ds(s * ROWS_PER_TILE,
                                              ROWS_PER_TILE)],
                        xs_sp.at[pl.ds(s * ROWS_PER_TILE, ROWS_PER_TILE)])
        plsc.subcore_barrier()

        def group(g, _):
            base = g * NBUF
            for b in range(NBUF):
                j = base + b

                @pl.when(g > 0)
                def _():
                    pltpu.make_async_copy(
                        rows_v.at[b], acc.at[dst_v.at[j - NBUF]],
                        ssem.at[b]).wait()
                pltpu.async_copy(xs_sp.at[src_v.at[j]], rows_v.at[b],
                                 gsem.at[b])
            for b in range(NBUF):
                j = base + b
                pltpu.make_async_copy(xs_sp.at[src_v.at[j]], rows_v.at[b],
                                      gsem.at[b]).wait()
                pltpu.async_copy(rows_v.at[b], acc.at[dst_v.at[j]],
                                 ssem.at[b], add=True)
            return 0
        lax.fori_loop(0, KS // NBUF, group, 0)
        for b in range(NBUF):
            pltpu.make_async_copy(rows_v.at[b],
                                  acc.at[dst_v.at[KS - NBUF + b]],
                                  ssem.at[b]).wait()
        plsc.subcore_barrier()

        pltpu.sync_copy(acc.at[pl.ds(s * ROWS_PER_TILE, ROWS_PER_TILE)],
                        out_hbm.at[qi].at[pl.ds(s * ROWS_PER_TILE,
                                                ROWS_PER_TILE)])
        plsc.subcore_barrier()
  return _sc_agg_body


@functools.lru_cache(maxsize=None)
def _make_sc_agg(d, feat_split):
    quad = d == 128
    dh = 32
    sup = 4
    nbuf = 2
    ks = K // sup
    zrows = 32
    nslab = 4 if quad else NC
    mesh = plsc.VectorSubcoreMesh(core_axis_name="c", subcore_axis_name="s")
    body = (_make_body_quad(sup, ks, zrows, nbuf) if quad
            else _make_body(sup, ks, zrows, True, nbuf, True))
    return pl.kernel(
        body,
        out_type=jax.ShapeDtypeStruct((nslab, NACC, dh), jnp.float32),
        mesh=mesh,
        compiler_params=pltpu.CompilerParams(use_tc_tiling_on_sc=False),
        scratch_types=[
            pltpu.VMEM((ks, sup * CHUNK), jnp.int32),
            pltpu.VMEM((ks, sup * CHUNK), jnp.int32),
            pltpu.VMEM((nbuf, sup * CHUNK, dh), jnp.float32),
            pltpu.VMEM((zrows, dh), jnp.float32),
            pltpu.VMEM_SHARED((NACC, dh), jnp.float32),
            pltpu.VMEM_SHARED((NACC, dh), jnp.float32),
            pltpu.SemaphoreType.DMA((nbuf,)),
            pltpu.SemaphoreType.DMA((nbuf,)),
        ],
    )


def _tc_layer_body(x_ref, agg_ref, w1_ref, b1_ref, w2_ref, b2_ref,
                   g_ref, be_ref, gi_ref, h_ref, pool_ref, *, feat_split):
    agg = jnp.concatenate(
        [agg_ref[q, :N_NODES, :] for q in range(agg_ref.shape[0])], axis=1)
    u = x_ref[...] + agg
    t = jnp.dot(u, w1_ref[...], preferred_element_type=jnp.float32)
    t = jnp.maximum(t + b1_ref[...], 0.0)
    v = jnp.dot(t, w2_ref[...], preferred_element_type=jnp.float32)
    v = jnp.maximum(v + b2_ref[...], 0.0)
    mean = jnp.mean(v, axis=0, keepdims=True)
    ctr = v - mean
    var = jnp.mean(ctr * ctr, axis=0, keepdims=True)
    hn = (v - mean) / jnp.sqrt(var + BN_EPS) * g_ref[...] + be_ref[...]
    h_ref[...] = hn
    onehot = (lax.broadcasted_iota(jnp.int32, (N_GRAPHS, N_NODES), 0)
              == gi_ref[...]).astype(jnp.float32)
    pool_ref[...] = jnp.dot(onehot, hn, preferred_element_type=jnp.float32,
                            precision=lax.Precision.HIGHEST)


def _tc_layer(x, agg, p, gi2d, feat_split):
    return pl.pallas_call(
        functools.partial(_tc_layer_body, feat_split=feat_split),
        out_shape=[
            jax.ShapeDtypeStruct((N_NODES, DIM), jnp.float32),
            jax.ShapeDtypeStruct((N_GRAPHS, DIM), jnp.float32),
        ],
    )(x, agg, p["W1"], p["b1"].reshape(1, -1), p["W2"],
      p["b2"].reshape(1, -1), p["gamma"].reshape(1, -1),
      p["beta"].reshape(1, -1), gi2d)


def kernel(node_features, edge_index, graph_index, params):
    pad = EPAD - N_EDGES
    src_flat = jnp.concatenate([edge_index[0], jnp.zeros((pad,), jnp.int32)])
    dst_flat = jnp.concatenate(
        [edge_index[1], jnp.full((pad,), N_NODES, jnp.int32)])
    gi2d = graph_index.reshape(1, N_NODES)

    x = node_features
    hs, pools = [], []
    for i in range(N_LAYERS):
        p = params[f"layer_{i}"]
        d = x.shape[1]
        src2d = src_flat.reshape(-1, 4 * CHUNK)
        dst2d = dst_flat.reshape(-1, 4 * CHUNK)
        nslab = d // 32
        xin = jnp.stack([x[:, 32 * q:32 * (q + 1)] for q in range(nslab)])
        xin = jnp.concatenate(
            [xin, jnp.zeros((nslab, NACC - N_NODES, 32), jnp.float32)],
            axis=1)
        agg = _make_sc_agg(d, True)(xin, src2d, dst2d)
        h, pool = _tc_layer(x, agg, p, gi2d, True)
        x = h
        hs.append(h)
        pools.append(pool)
    return jnp.concatenate(pools, axis=1), jnp.concatenate(hs, axis=1)
